# baseline (speedup 1.0000x reference)
import functools

import jax
import jax.numpy as jnp
from jax import lax
from jax.experimental import pallas as pl
from jax.experimental.pallas import tpu as pltpu
from jax.experimental.pallas import tpu_sc as plsc

B, T, DIM = 1024, 200, 64
_NW = 32
_ROWS_PER_W = B // _NW
_HALF = T // 2
_NV = DIM // 16


def _vreg_slices(ref, t):
    return [ref[t, pl.ds(16 * j, 16)] for j in range(_NV)]


def _fused_body(x_hbm, seg_hbm, table_hbm, segtab_hbm, pos_hbm, gamma_hbm,
                beta_hbm, out_hbm, idx_v, seg_v, rows_v, out_v, pos_v,
                segtab_v, gb_v, sem):
    wid = lax.axis_index("s") * 2 + lax.axis_index("c")

    pltpu.sync_copy(pos_hbm, pos_v)
    pltpu.sync_copy(segtab_hbm, segtab_v)
    pltpu.sync_copy(gamma_hbm, gb_v.at[0])
    pltpu.sync_copy(beta_hbm, gb_v.at[1])

    s0 = [segtab_v[0, pl.ds(16 * j, 16)] for j in range(_NV)]
    sd = [segtab_v[1, pl.ds(16 * j, 16)] - s0[j] for j in range(_NV)]
    gam = [gb_v[0, pl.ds(16 * j, 16)] for j in range(_NV)]
    bet = [gb_v[1, pl.ds(16 * j, 16)] for j in range(_NV)]

    def row_body(r, _):
        b = wid * _ROWS_PER_W + r
        pltpu.sync_copy(x_hbm.at[b], idx_v)
        pltpu.sync_copy(seg_hbm.at[b], seg_v.at[pl.ds(0, T)])
        cp0 = pltpu.async_copy(table_hbm.at[idx_v.at[0]],
                               rows_v.at[pl.ds(0, _HALF)], sem)
        cp1 = pltpu.async_copy(table_hbm.at[idx_v.at[1]],
                               rows_v.at[pl.ds(_HALF, _HALF)], sem)
        cp0.wait()
        cp1.wait()

        @plsc.parallel_loop(0, T, unroll=4)
        def token_body(t):
            f = seg_v[pl.ds(t, 16)][0].astype(jnp.float32)
            e = [rows_v[t, pl.ds(16 * j, 16)] + pos_v[t, pl.ds(16 * j, 16)]
                 + s0[j] + f * sd[j] for j in range(_NV)]
            tot = jnp.sum(e[0] + e[1] + e[2] + e[3])
            totq = jnp.sum(e[0] * e[0] + e[1] * e[1]
                           + e[2] * e[2] + e[3] * e[3])
            mean = tot * (1.0 / DIM)
            var = totq * (1.0 / DIM) - mean * mean
            v16 = jnp.full((16,), var + 1e-5, dtype=jnp.float32)
            seed = plsc.bitcast(
                jnp.int32(0x5F3759DF) - (plsc.bitcast(v16, jnp.int32) >> 1),
                jnp.float32)
            half = v16 * 0.5
            r0 = seed * (1.5 - half * seed * seed)
            r1 = r0 * (1.5 - half * r0 * r0)
            rs = r1 * (1.5 - half * r1 * r1)
            for j in range(_NV):
                gs = gam[j] * rs
                out_v[t, pl.ds(16 * j, 16)] = e[j] * gs + (bet[j] - mean * gs)

        pltpu.sync_copy(out_v, out_hbm.at[b])
        return ()

    lax.fori_loop(0, _ROWS_PER_W, row_body, ())


def kernel(x, segment, tok_table, seg_table, pos_table, gamma, beta):
    x3 = x.astype(jnp.int32).reshape(B, 2, _HALF)
    seg = segment.astype(jnp.int32)
    mesh = plsc.VectorSubcoreMesh(core_axis_name="c", subcore_axis_name="s")
    fused = pl.kernel(
        _fused_body,
        out_type=jax.ShapeDtypeStruct((B, T, DIM), jnp.float32),
        mesh=mesh,
        scratch_types=[
            pltpu.VMEM((2, _HALF), jnp.int32),
            pltpu.VMEM((T + 16, ), jnp.int32),
            pltpu.VMEM((T, DIM), jnp.float32),
            pltpu.VMEM((T, DIM), jnp.float32),
            pltpu.VMEM((T, DIM), jnp.float32),
            pltpu.VMEM((2, DIM), jnp.float32),
            pltpu.VMEM((2, DIM), jnp.float32),
            pltpu.SemaphoreType.DMA,
        ],
        compiler_params=pltpu.CompilerParams(use_tc_tiling_on_sc=False,
                                             needs_layout_passes=False),
    )
    return fused(x3, seg, tok_table, seg_table, pos_table[:T], gamma, beta)


# speedup vs baseline: 1.1071x; 1.1071x over previous
import jax
import jax.numpy as jnp
from jax import lax
from jax.experimental import pallas as pl
from jax.experimental.pallas import tpu as pltpu
from jax.experimental.pallas import tpu_sc as plsc

B, T, DIM = 1024, 200, 64
_NW = 32
_RPW = B // _NW
_HALF = T // 2
_NV = DIM // 16


def _fused_body(x_hbm, seg_hbm, table_hbm, segtab_hbm, pos_hbm, gamma_hbm,
                beta_hbm, out_hbm, idx_all, seg_all, rows_a, rows_b, out_a,
                out_b, pos_v, segtab_v, gb_v, sem_ga, sem_gb, sem_oa, sem_ob):
    wid = lax.axis_index("s") * 2 + lax.axis_index("c")
    b0 = wid * _RPW

    pltpu.sync_copy(pos_hbm, pos_v)
    pltpu.sync_copy(segtab_hbm, segtab_v)
    pltpu.sync_copy(gamma_hbm, gb_v.at[0])
    pltpu.sync_copy(beta_hbm, gb_v.at[1])
    pltpu.sync_copy(x_hbm.at[pl.ds(b0, _RPW)], idx_all)
    pltpu.sync_copy(seg_hbm.at[pl.ds(b0, _RPW)], seg_all.at[pl.ds(0, _RPW)])

    s0 = [segtab_v[0, pl.ds(16 * j, 16)] for j in range(_NV)]
    sd = [segtab_v[1, pl.ds(16 * j, 16)] - s0[j] for j in range(_NV)]
    gam = [gb_v[0, pl.ds(16 * j, 16)] for j in range(_NV)]
    bet = [gb_v[1, pl.ds(16 * j, 16)] for j in range(_NV)]

    def gather_cps(r, rows_buf, sem):
        return (pltpu.make_async_copy(table_hbm.at[idx_all.at[r, 0]],
                                      rows_buf.at[pl.ds(0, _HALF)], sem),
                pltpu.make_async_copy(table_hbm.at[idx_all.at[r, 1]],
                                      rows_buf.at[pl.ds(_HALF, _HALF)], sem))

    def fire_gather(r, rows_buf, sem):
        for cp in gather_cps(r, rows_buf, sem):
            cp.start()

    def wait_gather(r, rows_buf, sem):
        for cp in gather_cps(r, rows_buf, sem):
            cp.wait()

    def compute_row(r, rows_buf, out_buf):
        @plsc.parallel_loop(0, T, unroll=4)
        def token_body(t):
            f = seg_all[r, pl.ds(t, 16)][0].astype(jnp.float32)
            e = [rows_buf[t, pl.ds(16 * j, 16)] + pos_v[t, pl.ds(16 * j, 16)]
                 + s0[j] + f * sd[j] for j in range(_NV)]
            tot = jnp.sum(e[0] + e[1] + e[2] + e[3])
            totq = jnp.sum(e[0] * e[0] + e[1] * e[1]
                           + e[2] * e[2] + e[3] * e[3])
            mean = tot * (1.0 / DIM)
            var = totq * (1.0 / DIM) - mean * mean
            v16 = jnp.full((16,), var + 1e-5, dtype=jnp.float32)
            seed = plsc.bitcast(
                jnp.int32(0x5F3759DF) - (plsc.bitcast(v16, jnp.int32) >> 1),
                jnp.float32)
            half = v16 * 0.5
            r0 = seed * (1.5 - half * seed * seed)
            rs = r0 * (1.5 - half * r0 * r0)
            for j in range(_NV):
                gs = gam[j] * rs
                out_buf[t, pl.ds(16 * j, 16)] = e[j] * gs + (bet[j] - mean * gs)

    def process(i, r, rows_buf, out_buf, sem_g, sem_o):
        wait_gather(r, rows_buf, sem_g)

        @pl.when(i > 0)
        def _():
            pltpu.make_async_copy(out_buf, out_hbm.at[b0 + r - 2],
                                  sem_o).wait()

        compute_row(r, rows_buf, out_buf)
        pltpu.async_copy(out_buf, out_hbm.at[b0 + r], sem_o)

        @pl.when(r + 2 < _RPW)
        def _():
            fire_gather(r + 2, rows_buf, sem_g)

    fire_gather(0, rows_a, sem_ga)
    fire_gather(1, rows_b, sem_gb)

    def pair_body(i, _):
        process(i, 2 * i, rows_a, out_a, sem_ga, sem_oa)
        process(i, 2 * i + 1, rows_b, out_b, sem_gb, sem_ob)
        return ()

    lax.fori_loop(0, _RPW // 2, pair_body, ())

    pltpu.make_async_copy(out_a, out_hbm.at[b0 + _RPW - 2], sem_oa).wait()
    pltpu.make_async_copy(out_b, out_hbm.at[b0 + _RPW - 1], sem_ob).wait()


def kernel(x, segment, tok_table, seg_table, pos_table, gamma, beta):
    x3 = x.astype(jnp.int32).reshape(B, 2, _HALF)
    seg = segment.astype(jnp.int32)
    mesh = plsc.VectorSubcoreMesh(core_axis_name="c", subcore_axis_name="s")
    fused = pl.kernel(
        _fused_body,
        out_type=jax.ShapeDtypeStruct((B, T, DIM), jnp.float32),
        mesh=mesh,
        scratch_types=[
            pltpu.VMEM((_RPW, 2, _HALF), jnp.int32),
            pltpu.VMEM((_RPW + 1, T), jnp.int32),
            pltpu.VMEM((T, DIM), jnp.float32),
            pltpu.VMEM((T, DIM), jnp.float32),
            pltpu.VMEM((T, DIM), jnp.float32),
            pltpu.VMEM((T, DIM), jnp.float32),
            pltpu.VMEM((T, DIM), jnp.float32),
            pltpu.VMEM((2, DIM), jnp.float32),
            pltpu.VMEM((2, DIM), jnp.float32),
            pltpu.SemaphoreType.DMA,
            pltpu.SemaphoreType.DMA,
            pltpu.SemaphoreType.DMA,
            pltpu.SemaphoreType.DMA,
        ],
        compiler_params=pltpu.CompilerParams(use_tc_tiling_on_sc=False,
                                             needs_layout_passes=False),
    )
    return fused(x3, seg, tok_table, seg_table, pos_table[:T], gamma, beta)


# speedup vs baseline: 1.4715x; 1.3292x over previous
import jax
import jax.numpy as jnp
from jax import lax
from jax.experimental import pallas as pl
from jax.experimental.pallas import tpu as pltpu
from jax.experimental.pallas import tpu_sc as plsc

B, T, DIM = 1024, 200, 64
_NW = 32
_RPW = B // _NW
_HALF = T // 2
_NV = DIM // 16


def _fused_body(x_hbm, seg_hbm, table_hbm, segtab_hbm, pos_hbm, gamma_hbm,
                beta_hbm, out_hbm, idx_all, seg_all, rows_a, rows_b, out_a,
                out_b, pos_v, segtab_v, gb_v, sem_ga, sem_gb, sem_oa, sem_ob):
    wid = lax.axis_index("s") * 2 + lax.axis_index("c")
    b0 = wid * _RPW

    pltpu.sync_copy(pos_hbm, pos_v)
    pltpu.sync_copy(segtab_hbm, segtab_v)
    pltpu.sync_copy(gamma_hbm, gb_v.at[0])
    pltpu.sync_copy(beta_hbm, gb_v.at[1])
    pltpu.sync_copy(x_hbm.at[pl.ds(b0, _RPW)], idx_all)
    pltpu.sync_copy(seg_hbm.at[pl.ds(b0, _RPW)], seg_all.at[pl.ds(0, _RPW)])

    s0 = [segtab_v[0, pl.ds(16 * j, 16)] for j in range(_NV)]
    sd = [segtab_v[1, pl.ds(16 * j, 16)] - s0[j] for j in range(_NV)]
    gam = [gb_v[0, pl.ds(16 * j, 16)] for j in range(_NV)]
    bet = [gb_v[1, pl.ds(16 * j, 16)] for j in range(_NV)]

    @plsc.parallel_loop(0, T, unroll=2)
    def _posadd(t):
        for j in range(_NV):
            pos_v[t, pl.ds(16 * j, 16)] = pos_v[t, pl.ds(16 * j, 16)] + s0[j]

    _dn = lax.GatherDimensionNumbers(offset_dims=(), collapsed_slice_dims=(0,),
                                     start_index_map=(0,))
    _lane0 = jnp.zeros((16, 1), jnp.int32)
    _lane15 = jnp.full((16, 1), 15, jnp.int32)

    def _bcast(v, lane_idx):
        return lax.gather(v, lane_idx, _dn, slice_sizes=(1,),
                          mode=lax.GatherScatterMode.PROMISE_IN_BOUNDS)

    def gather_cps(r, rows_buf, sem):
        return (pltpu.make_async_copy(table_hbm.at[idx_all.at[r, 0]],
                                      rows_buf.at[pl.ds(0, _HALF)], sem),
                pltpu.make_async_copy(table_hbm.at[idx_all.at[r, 1]],
                                      rows_buf.at[pl.ds(_HALF, _HALF)], sem))

    def fire_gather(r, rows_buf, sem):
        for cp in gather_cps(r, rows_buf, sem):
            cp.start()

    def wait_gather(r, rows_buf, sem):
        for cp in gather_cps(r, rows_buf, sem):
            cp.wait()

    def compute_row(r, rows_buf, out_buf):
        @plsc.parallel_loop(0, T, unroll=4)
        def token_body(t):
            f = _bcast(seg_all[r, pl.ds(t, 16)].astype(jnp.float32), _lane0)
            e = [rows_buf[t, pl.ds(16 * j, 16)] + pos_v[t, pl.ds(16 * j, 16)]
                 + f * sd[j] for j in range(_NV)]
            tot = _bcast(plsc.cumsum(e[0] + e[1] + e[2] + e[3]), _lane15)
            totq = _bcast(plsc.cumsum(e[0] * e[0] + e[1] * e[1]
                                      + e[2] * e[2] + e[3] * e[3]), _lane15)
            mean = tot * (1.0 / DIM)
            v16 = totq * (1.0 / DIM) - mean * mean + 1e-5
            seed = plsc.bitcast(
                jnp.int32(0x5F3759DF) - (plsc.bitcast(v16, jnp.int32) >> 1),
                jnp.float32)
            half = v16 * 0.5
            r0 = seed * (1.5 - half * seed * seed)
            rs = r0 * (1.5 - half * r0 * r0)
            mrs = mean * rs
            for j in range(_NV):
                gs = gam[j] * rs
                out_buf[t, pl.ds(16 * j, 16)] = e[j] * gs + (bet[j] - mrs * gam[j])

    def process(i, r, rows_buf, out_buf, sem_g, sem_o):
        wait_gather(r, rows_buf, sem_g)

        @pl.when(i > 0)
        def _():
            pltpu.make_async_copy(out_buf, out_hbm.at[b0 + r - 2],
                                  sem_o).wait()

        compute_row(r, rows_buf, out_buf)
        pltpu.async_copy(out_buf, out_hbm.at[b0 + r], sem_o)

        @pl.when(r + 2 < _RPW)
        def _():
            fire_gather(r + 2, rows_buf, sem_g)

    fire_gather(0, rows_a, sem_ga)
    fire_gather(1, rows_b, sem_gb)

    def pair_body(i, _):
        process(i, 2 * i, rows_a, out_a, sem_ga, sem_oa)
        process(i, 2 * i + 1, rows_b, out_b, sem_gb, sem_ob)
        return ()

    lax.fori_loop(0, _RPW // 2, pair_body, ())

    pltpu.make_async_copy(out_a, out_hbm.at[b0 + _RPW - 2], sem_oa).wait()
    pltpu.make_async_copy(out_b, out_hbm.at[b0 + _RPW - 1], sem_ob).wait()


def kernel(x, segment, tok_table, seg_table, pos_table, gamma, beta):
    x3 = x.astype(jnp.int32).reshape(B, 2, _HALF)
    seg = segment.astype(jnp.int32)
    mesh = plsc.VectorSubcoreMesh(core_axis_name="c", subcore_axis_name="s")
    fused = pl.kernel(
        _fused_body,
        out_type=jax.ShapeDtypeStruct((B, T, DIM), jnp.float32),
        mesh=mesh,
        scratch_types=[
            pltpu.VMEM((_RPW, 2, _HALF), jnp.int32),
            pltpu.VMEM((_RPW + 1, T), jnp.int32),
            pltpu.VMEM((T, DIM), jnp.float32),
            pltpu.VMEM((T, DIM), jnp.float32),
            pltpu.VMEM((T, DIM), jnp.float32),
            pltpu.VMEM((T, DIM), jnp.float32),
            pltpu.VMEM((T, DIM), jnp.float32),
            pltpu.VMEM((2, DIM), jnp.float32),
            pltpu.VMEM((2, DIM), jnp.float32),
            pltpu.SemaphoreType.DMA,
            pltpu.SemaphoreType.DMA,
            pltpu.SemaphoreType.DMA,
            pltpu.SemaphoreType.DMA,
        ],
        compiler_params=pltpu.CompilerParams(use_tc_tiling_on_sc=False,
                                             needs_layout_passes=False),
    )
    return fused(x3, seg, tok_table, seg_table, pos_table[:T], gamma, beta)
